# baseline (speedup 1.0000x reference)
import functools

import jax
import jax.numpy as jnp
from jax import lax
from jax.experimental import pallas as pl
from jax.experimental.pallas import tpu as pltpu
from jax.experimental.pallas import tpu_sc as plsc

_B = 16384
_D = 128
_NC = 2
_NS = 16
_NW = _NC * _NS
_BPW = _B // _NW
_CH = 128
_NCHUNK = _BPW // _CH
_LANES = 8


def _rsqrt(x):
    i = lax.bitcast_convert_type(x, jnp.int32)
    i = jnp.int32(0x5F3759DF) - lax.shift_right_logical(i, 1)
    y = lax.bitcast_convert_type(i, jnp.float32)
    for _ in range(3):
        y = y * (jnp.float32(1.5) - jnp.float32(0.5) * x * y * y)
    return y


def _trans_d_body(heads, rels, invs, ent, entp, rel2, relp, out,
                  idxh, idxr, idxe, hbuf, hpbuf, rbuf, rpbuf, obuf, sem):
    wid = lax.axis_index("s") * _NC + lax.axis_index("c")
    base = wid * _BPW

    def run_chunk(c, _):
        off = base + c * _CH
        pltpu.sync_copy(heads.at[pl.ds(off, _CH)], idxh)
        pltpu.sync_copy(rels.at[pl.ds(off, _CH)], idxr)
        pltpu.sync_copy(invs.at[pl.ds(off, _CH)], idxe)
        for k in range(_CH // 16):
            s = pl.ds(k * 16, 16)
            idxe[s] = idxr[s] + jnp.int32(1000) * idxe[s]
        cps = [
            pltpu.async_copy(ent.at[idxh], hbuf, sem),
            pltpu.async_copy(entp.at[idxh], hpbuf, sem),
            pltpu.async_copy(rel2.at[idxe], rbuf, sem),
            pltpu.async_copy(relp.at[idxr], rpbuf, sem),
        ]
        for cp in cps:
            cp.wait()

        @plsc.parallel_loop(0, _CH, unroll=8)
        def row(i):
            hv = [hbuf[i, pl.ds(d * 16, 16)] for d in range(_LANES)]
            hpv = [hpbuf[i, pl.ds(d * 16, 16)] for d in range(_LANES)]
            acc = hv[0] * hpv[0]
            for d in range(1, _LANES):
                acc = acc + hv[d] * hpv[d]
            inner = jnp.sum(acc)
            tv = [rpbuf[i, pl.ds(d * 16, 16)] * inner + hv[d]
                  for d in range(_LANES)]
            nacc = tv[0] * tv[0]
            for d in range(1, _LANES):
                nacc = nacc + tv[d] * tv[d]
            inv1 = _rsqrt(jnp.maximum(jnp.sum(nacc), jnp.float32(1e-24)))
            uv = [tv[d] * inv1 + rbuf[i, pl.ds(d * 16, 16)]
                  for d in range(_LANES)]
            n2 = uv[0] * uv[0]
            for d in range(1, _LANES):
                n2 = n2 + uv[d] * uv[d]
            inv2 = _rsqrt(jnp.maximum(jnp.sum(n2), jnp.float32(1e-24)))
            for d in range(_LANES):
                obuf[i, pl.ds(d * 16, 16)] = uv[d] * inv2

        pltpu.sync_copy(obuf, out.at[pl.ds(off, _CH)])
        return 0

    lax.fori_loop(0, _NCHUNK, run_chunk, 0)


@functools.partial(jax.jit, donate_argnums=())
def _trans_d(heads_i32, rels_i32, inv_i32, entity_emb, entity_emb_p,
             rel2, rel_emb_p):
    mesh = plsc.VectorSubcoreMesh(
        core_axis_name="c", subcore_axis_name="s",
        num_cores=_NC, num_subcores=_NS)
    return pl.kernel(
        _trans_d_body,
        out_type=jax.ShapeDtypeStruct((_B, _D), jnp.float32),
        mesh=mesh,
        compiler_params=pltpu.CompilerParams(needs_layout_passes=False),
        scratch_types=[
            pltpu.VMEM((_CH,), jnp.int32),
            pltpu.VMEM((_CH,), jnp.int32),
            pltpu.VMEM((_CH,), jnp.int32),
            pltpu.VMEM((_CH, _D), jnp.float32),
            pltpu.VMEM((_CH, _D), jnp.float32),
            pltpu.VMEM((_CH, _D), jnp.float32),
            pltpu.VMEM((_CH, _D), jnp.float32),
            pltpu.VMEM((_CH, _D), jnp.float32),
            pltpu.SemaphoreType.DMA,
        ],
    )(heads_i32, rels_i32, inv_i32, entity_emb, entity_emb_p,
      rel2, rel_emb_p)


def kernel(heads, relations, inverse, entity_emb, entity_emb_p,
           rel_emb, rel_emb_p):
    heads_i32 = heads.astype(jnp.int32)
    rels_i32 = relations.astype(jnp.int32)
    inv_i32 = inverse.astype(jnp.int32)
    rel2 = jnp.concatenate([rel_emb, -rel_emb], axis=0)
    return _trans_d(heads_i32, rels_i32, inv_i32, entity_emb,
                    entity_emb_p, rel2, rel_emb_p)


# speedup vs baseline: 1.2104x; 1.2104x over previous
import functools

import jax
import jax.numpy as jnp
from jax import lax
from jax.experimental import pallas as pl
from jax.experimental.pallas import tpu as pltpu
from jax.experimental.pallas import tpu_sc as plsc

_B = 16384
_D = 128
_NC = 2
_NS = 16
_NW = _NC * _NS
_BPW = _B // _NW
_CH = 128
_NCHUNK = _BPW // _CH
_LANES = 8


def _rsqrt(x):
    i = lax.bitcast_convert_type(x, jnp.int32)
    i = jnp.int32(0x5F3759DF) - lax.shift_right_logical(i, 1)
    y = lax.bitcast_convert_type(i, jnp.float32)
    for _ in range(3):
        y = y * (jnp.float32(1.5) - jnp.float32(0.5) * x * y * y)
    return y


def _trans_d_body(heads, rels, invs, ent, entp, rel2, relp, out,
                  idxh, idxr, idxe, hbuf, hpbuf, rbuf, rpbuf, obuf, sem):
    wid = lax.axis_index("s") * _NC + lax.axis_index("c")
    base = wid * _BPW

    def run_chunk(c, _):
        off = base + c * _CH
        pltpu.sync_copy(heads.at[pl.ds(off, _CH)], idxh)
        pltpu.sync_copy(rels.at[pl.ds(off, _CH)], idxr)
        pltpu.sync_copy(invs.at[pl.ds(off, _CH)], idxe)
        for k in range(_CH // 16):
            s = pl.ds(k * 16, 16)
            idxe[s] = idxr[s] + jnp.int32(1000) * idxe[s]
        cps = [
            pltpu.async_copy(ent.at[idxh], hbuf, sem),
            pltpu.async_copy(entp.at[idxh], hpbuf, sem),
            pltpu.async_copy(rel2.at[idxe], rbuf, sem),
            pltpu.async_copy(relp.at[idxr], rpbuf, sem),
        ]
        for cp in cps:
            cp.wait()

        @plsc.parallel_loop(0, _CH, unroll=4)
        def row(i):
            hv = [hbuf[i, pl.ds(d * 16, 16)] for d in range(_LANES)]
            hpv = [hpbuf[i, pl.ds(d * 16, 16)] for d in range(_LANES)]
            acc = hv[0] * hpv[0]
            for d in range(1, _LANES):
                acc = acc + hv[d] * hpv[d]
            inner = jnp.sum(acc)
            tv = [rpbuf[i, pl.ds(d * 16, 16)] * inner + hv[d]
                  for d in range(_LANES)]
            nacc = tv[0] * tv[0]
            for d in range(1, _LANES):
                nacc = nacc + tv[d] * tv[d]
            inv1 = _rsqrt(jnp.maximum(jnp.sum(nacc), jnp.float32(1e-24)))
            uv = [tv[d] * inv1 + rbuf[i, pl.ds(d * 16, 16)]
                  for d in range(_LANES)]
            n2 = uv[0] * uv[0]
            for d in range(1, _LANES):
                n2 = n2 + uv[d] * uv[d]
            inv2 = _rsqrt(jnp.maximum(jnp.sum(n2), jnp.float32(1e-24)))
            for d in range(_LANES):
                obuf[i, pl.ds(d * 16, 16)] = uv[d] * inv2

        pltpu.sync_copy(obuf, out.at[pl.ds(off, _CH)])
        return 0

    lax.fori_loop(0, _NCHUNK, run_chunk, 0)


@functools.partial(jax.jit, donate_argnums=())
def _trans_d(heads_i32, rels_i32, inv_i32, entity_emb, entity_emb_p,
             rel2, rel_emb_p):
    mesh = plsc.VectorSubcoreMesh(
        core_axis_name="c", subcore_axis_name="s",
        num_cores=_NC, num_subcores=_NS)
    return pl.kernel(
        _trans_d_body,
        out_type=jax.ShapeDtypeStruct((_B, _D), jnp.float32),
        mesh=mesh,
        compiler_params=pltpu.CompilerParams(needs_layout_passes=False),
        scratch_types=[
            pltpu.VMEM((_CH,), jnp.int32),
            pltpu.VMEM((_CH,), jnp.int32),
            pltpu.VMEM((_CH,), jnp.int32),
            pltpu.VMEM((_CH, _D), jnp.float32),
            pltpu.VMEM((_CH, _D), jnp.float32),
            pltpu.VMEM((_CH, _D), jnp.float32),
            pltpu.VMEM((_CH, _D), jnp.float32),
            pltpu.VMEM((_CH, _D), jnp.float32),
            pltpu.SemaphoreType.DMA,
        ],
    )(heads_i32, rels_i32, inv_i32, entity_emb, entity_emb_p,
      rel2, rel_emb_p)


def kernel(heads, relations, inverse, entity_emb, entity_emb_p,
           rel_emb, rel_emb_p):
    heads_i32 = heads.astype(jnp.int32)
    rels_i32 = relations.astype(jnp.int32)
    inv_i32 = inverse.astype(jnp.int32)
    rel2 = jnp.concatenate([rel_emb, -rel_emb], axis=0)
    return _trans_d(heads_i32, rels_i32, inv_i32, entity_emb,
                    entity_emb_p, rel2, rel_emb_p)


# speedup vs baseline: 1.2820x; 1.0592x over previous
import functools

import jax
import jax.numpy as jnp
from jax import lax
from jax.experimental import pallas as pl
from jax.experimental.pallas import tpu as pltpu
from jax.experimental.pallas import tpu_sc as plsc

_B = 16384
_D = 128
_NC = 2
_NS = 16
_NW = _NC * _NS
_BPW = _B // _NW
_CH = 128
_NCHUNK = _BPW // _CH
_LANES = 8


def _rsqrt(x):
    i = lax.bitcast_convert_type(x, jnp.int32)
    i = jnp.int32(0x5F3759DF) - lax.shift_right_logical(i, 1)
    y = lax.bitcast_convert_type(i, jnp.float32)
    for _ in range(3):
        y = y * (jnp.float32(1.5) - jnp.float32(0.5) * x * y * y)
    return y


def _trans_d_body(heads, rels, invs, ent, entp, rel2, relp, out,
                  idxh, idxr, idxe, hbuf, hpbuf, rbuf, rpbuf, obuf, sem):
    wid = lax.axis_index("s") * _NC + lax.axis_index("c")
    base = wid * _BPW

    pltpu.sync_copy(heads.at[pl.ds(base, _BPW)], idxh)
    pltpu.sync_copy(rels.at[pl.ds(base, _BPW)], idxr)
    pltpu.sync_copy(invs.at[pl.ds(base, _BPW)], idxe)

    @plsc.parallel_loop(0, _BPW // 16, unroll=4)
    def _mix(k):
        s = pl.ds(k * 16, 16)
        idxe[s] = idxr[s] + jnp.int32(1000) * idxe[s]

    def run_chunk(c, _):
        off = base + c * _CH
        cs = pl.ds(c * _CH, _CH)
        cps = [
            pltpu.async_copy(ent.at[idxh.at[cs]], hbuf, sem),
            pltpu.async_copy(entp.at[idxh.at[cs]], hpbuf, sem),
            pltpu.async_copy(rel2.at[idxe.at[cs]], rbuf, sem),
            pltpu.async_copy(relp.at[idxr.at[cs]], rpbuf, sem),
        ]
        for cp in cps:
            cp.wait()

        @plsc.parallel_loop(0, _CH, unroll=4)
        def row(i):
            hv = [hbuf[i, pl.ds(d * 16, 16)] for d in range(_LANES)]
            hpv = [hpbuf[i, pl.ds(d * 16, 16)] for d in range(_LANES)]
            acc = hv[0] * hpv[0]
            for d in range(1, _LANES):
                acc = acc + hv[d] * hpv[d]
            inner = jnp.sum(acc)
            tv = [rpbuf[i, pl.ds(d * 16, 16)] * inner + hv[d]
                  for d in range(_LANES)]
            nacc = tv[0] * tv[0]
            for d in range(1, _LANES):
                nacc = nacc + tv[d] * tv[d]
            inv1 = _rsqrt(jnp.maximum(jnp.sum(nacc), jnp.float32(1e-24)))
            uv = [tv[d] * inv1 + rbuf[i, pl.ds(d * 16, 16)]
                  for d in range(_LANES)]
            n2 = uv[0] * uv[0]
            for d in range(1, _LANES):
                n2 = n2 + uv[d] * uv[d]
            inv2 = _rsqrt(jnp.maximum(jnp.sum(n2), jnp.float32(1e-24)))
            for d in range(_LANES):
                obuf[i, pl.ds(d * 16, 16)] = uv[d] * inv2

        pltpu.sync_copy(obuf, out.at[pl.ds(off, _CH)])
        return 0

    lax.fori_loop(0, _NCHUNK, run_chunk, 0)


@functools.partial(jax.jit, donate_argnums=())
def _trans_d(heads_i32, rels_i32, inv_i32, entity_emb, entity_emb_p,
             rel2, rel_emb_p):
    mesh = plsc.VectorSubcoreMesh(
        core_axis_name="c", subcore_axis_name="s",
        num_cores=_NC, num_subcores=_NS)
    return pl.kernel(
        _trans_d_body,
        out_type=jax.ShapeDtypeStruct((_B, _D), jnp.float32),
        mesh=mesh,
        compiler_params=pltpu.CompilerParams(needs_layout_passes=False),
        scratch_types=[
            pltpu.VMEM((_BPW,), jnp.int32),
            pltpu.VMEM((_BPW,), jnp.int32),
            pltpu.VMEM((_BPW,), jnp.int32),
            pltpu.VMEM((_CH, _D), jnp.float32),
            pltpu.VMEM((_CH, _D), jnp.float32),
            pltpu.VMEM((_CH, _D), jnp.float32),
            pltpu.VMEM((_CH, _D), jnp.float32),
            pltpu.VMEM((_CH, _D), jnp.float32),
            pltpu.SemaphoreType.DMA,
        ],
    )(heads_i32, rels_i32, inv_i32, entity_emb, entity_emb_p,
      rel2, rel_emb_p)


def kernel(heads, relations, inverse, entity_emb, entity_emb_p,
           rel_emb, rel_emb_p):
    heads_i32 = heads.astype(jnp.int32)
    rels_i32 = relations.astype(jnp.int32)
    inv_i32 = inverse.astype(jnp.int32)
    rel2 = jnp.concatenate([rel_emb, -rel_emb], axis=0)
    return _trans_d(heads_i32, rels_i32, inv_i32, entity_emb,
                    entity_emb_p, rel2, rel_emb_p)


# speedup vs baseline: 1.5013x; 1.1710x over previous
import functools

import jax
import jax.numpy as jnp
from jax import lax
from jax.experimental import pallas as pl
from jax.experimental.pallas import tpu as pltpu
from jax.experimental.pallas import tpu_sc as plsc

_B = 16384
_D = 128
_NC = 2
_NS = 16
_NW = _NC * _NS
_BPW = _B // _NW
_CH = 64
_NCHUNK = _BPW // _CH
_NPAIR = _NCHUNK // 2
_LANES = 8


def _rsqrt(x):
    i = lax.bitcast_convert_type(x, jnp.int32)
    i = jnp.int32(0x5F3759DF) - lax.shift_right_logical(i, 1)
    y = lax.bitcast_convert_type(i, jnp.float32)
    for _ in range(3):
        y = y * (jnp.float32(1.5) - jnp.float32(0.5) * x * y * y)
    return y


def _trans_d_body(heads, rels, invs, ent, entp, rel2, relp, out,
                  idxh, idxr, idxe,
                  hbufA, hpbufA, rbufA, rpbufA, obufA,
                  hbufB, hpbufB, rbufB, rpbufB, obufB,
                  semA, semB, wsemA, wsemB):
    wid = lax.axis_index("s") * _NC + lax.axis_index("c")
    base = wid * _BPW

    pltpu.sync_copy(heads.at[pl.ds(base, _BPW)], idxh)
    pltpu.sync_copy(rels.at[pl.ds(base, _BPW)], idxr)
    pltpu.sync_copy(invs.at[pl.ds(base, _BPW)], idxe)

    @plsc.parallel_loop(0, _BPW // 16, unroll=4)
    def _mix(k):
        s = pl.ds(k * 16, 16)
        idxe[s] = idxr[s] + jnp.int32(1000) * idxe[s]

    def fire(c, hb, hpb, rb, rpb, sem):
        cs = pl.ds(c * _CH, _CH)
        pltpu.async_copy(ent.at[idxh.at[cs]], hb, sem)
        pltpu.async_copy(entp.at[idxh.at[cs]], hpb, sem)
        pltpu.async_copy(rel2.at[idxe.at[cs]], rb, sem)
        pltpu.async_copy(relp.at[idxr.at[cs]], rpb, sem)

    def drain(c, hb, hpb, rb, rpb, sem):
        cs = pl.ds(c * _CH, _CH)
        pltpu.make_async_copy(ent.at[idxh.at[cs]], hb, sem).wait()
        pltpu.make_async_copy(entp.at[idxh.at[cs]], hpb, sem).wait()
        pltpu.make_async_copy(rel2.at[idxe.at[cs]], rb, sem).wait()
        pltpu.make_async_copy(relp.at[idxr.at[cs]], rpb, sem).wait()

    def compute(hb, hpb, rpb, rb, ob):
        @plsc.parallel_loop(0, _CH, unroll=4)
        def row(i):
            hv = [hb[i, pl.ds(d * 16, 16)] for d in range(_LANES)]
            hpv = [hpb[i, pl.ds(d * 16, 16)] for d in range(_LANES)]
            acc = hv[0] * hpv[0]
            for d in range(1, _LANES):
                acc = acc + hv[d] * hpv[d]
            inner = jnp.sum(acc)
            tv = [rpb[i, pl.ds(d * 16, 16)] * inner + hv[d]
                  for d in range(_LANES)]
            nacc = tv[0] * tv[0]
            for d in range(1, _LANES):
                nacc = nacc + tv[d] * tv[d]
            inv1 = _rsqrt(jnp.maximum(jnp.sum(nacc), jnp.float32(1e-24)))
            uv = [tv[d] * inv1 + rb[i, pl.ds(d * 16, 16)]
                  for d in range(_LANES)]
            n2 = uv[0] * uv[0]
            for d in range(1, _LANES):
                n2 = n2 + uv[d] * uv[d]
            inv2 = _rsqrt(jnp.maximum(jnp.sum(n2), jnp.float32(1e-24)))
            for d in range(_LANES):
                ob[i, pl.ds(d * 16, 16)] = uv[d] * inv2

    fire(0, hbufA, hpbufA, rbufA, rpbufA, semA)

    def pair(p, _):
        c0 = 2 * p
        fire(c0 + 1, hbufB, hpbufB, rbufB, rpbufB, semB)
        drain(c0, hbufA, hpbufA, rbufA, rpbufA, semA)
        compute(hbufA, hpbufA, rpbufA, rbufA, obufA)
        wbA = pltpu.async_copy(
            obufA, out.at[pl.ds(base + c0 * _CH, _CH)], wsemA)

        @pl.when(p < _NPAIR - 1)
        def _():
            fire(c0 + 2, hbufA, hpbufA, rbufA, rpbufA, semA)

        drain(c0 + 1, hbufB, hpbufB, rbufB, rpbufB, semB)
        compute(hbufB, hpbufB, rpbufB, rbufB, obufB)
        wbB = pltpu.async_copy(
            obufB, out.at[pl.ds(base + (c0 + 1) * _CH, _CH)], wsemB)
        wbA.wait()
        wbB.wait()
        return 0

    lax.fori_loop(0, _NPAIR, pair, 0)


@functools.partial(jax.jit, donate_argnums=())
def _trans_d(heads_i32, rels_i32, inv_i32, entity_emb, entity_emb_p,
             rel2, rel_emb_p):
    mesh = plsc.VectorSubcoreMesh(
        core_axis_name="c", subcore_axis_name="s",
        num_cores=_NC, num_subcores=_NS)
    buf = lambda: pltpu.VMEM((_CH, _D), jnp.float32)
    return pl.kernel(
        _trans_d_body,
        out_type=jax.ShapeDtypeStruct((_B, _D), jnp.float32),
        mesh=mesh,
        compiler_params=pltpu.CompilerParams(needs_layout_passes=False),
        scratch_types=[
            pltpu.VMEM((_BPW,), jnp.int32),
            pltpu.VMEM((_BPW,), jnp.int32),
            pltpu.VMEM((_BPW,), jnp.int32),
            buf(), buf(), buf(), buf(), buf(),
            buf(), buf(), buf(), buf(), buf(),
            pltpu.SemaphoreType.DMA,
            pltpu.SemaphoreType.DMA,
            pltpu.SemaphoreType.DMA,
            pltpu.SemaphoreType.DMA,
        ],
    )(heads_i32, rels_i32, inv_i32, entity_emb, entity_emb_p,
      rel2, rel_emb_p)


def kernel(heads, relations, inverse, entity_emb, entity_emb_p,
           rel_emb, rel_emb_p):
    heads_i32 = heads.astype(jnp.int32)
    rels_i32 = relations.astype(jnp.int32)
    inv_i32 = inverse.astype(jnp.int32)
    rel2 = jnp.concatenate([rel_emb, -rel_emb], axis=0)
    return _trans_d(heads_i32, rels_i32, inv_i32, entity_emb,
                    entity_emb_p, rel2, rel_emb_p)


# speedup vs baseline: 1.6035x; 1.0681x over previous
import functools

import jax
import jax.numpy as jnp
from jax import lax
from jax.experimental import pallas as pl
from jax.experimental.pallas import tpu as pltpu
from jax.experimental.pallas import tpu_sc as plsc

_B = 16384
_D = 128
_NC = 2
_NS = 16
_NW = _NC * _NS
_BPW = _B // _NW
_CH = 64
_NCHUNK = _BPW // _CH
_NPAIR = _NCHUNK // 2
_LANES = 8


def _rsqrt(x):
    i = lax.bitcast_convert_type(x, jnp.int32)
    i = jnp.int32(0x5F3759DF) - lax.shift_right_logical(i, 1)
    y = lax.bitcast_convert_type(i, jnp.float32)
    for _ in range(2):
        y = y * (jnp.float32(1.5) - jnp.float32(0.5) * x * y * y)
    return y


def _trans_d_body(heads, rels, invs, ent, entp, rel2, relp, out,
                  idxh, idxr, idxe,
                  hbufA, hpbufA, rbufA, rpbufA, obufA,
                  hbufB, hpbufB, rbufB, rpbufB, obufB,
                  semA, semB, wsemA, wsemB):
    wid = lax.axis_index("s") * _NC + lax.axis_index("c")
    base = wid * _BPW

    pltpu.sync_copy(heads.at[pl.ds(base, _BPW)], idxh)
    pltpu.sync_copy(rels.at[pl.ds(base, _BPW)], idxr)
    pltpu.sync_copy(invs.at[pl.ds(base, _BPW)], idxe)

    @plsc.parallel_loop(0, _BPW // 16, unroll=4)
    def _mix(k):
        s = pl.ds(k * 16, 16)
        idxe[s] = idxr[s] + jnp.int32(1000) * idxe[s]

    def fire(c, hb, hpb, rb, rpb, sem):
        cs = pl.ds(c * _CH, _CH)
        pltpu.async_copy(ent.at[idxh.at[cs]], hb, sem)
        pltpu.async_copy(entp.at[idxh.at[cs]], hpb, sem)
        pltpu.async_copy(rel2.at[idxe.at[cs]], rb, sem)
        pltpu.async_copy(relp.at[idxr.at[cs]], rpb, sem)

    def drain(c, hb, hpb, rb, rpb, sem):
        cs = pl.ds(c * _CH, _CH)
        pltpu.make_async_copy(ent.at[idxh.at[cs]], hb, sem).wait()
        pltpu.make_async_copy(entp.at[idxh.at[cs]], hpb, sem).wait()
        pltpu.make_async_copy(rel2.at[idxe.at[cs]], rb, sem).wait()
        pltpu.make_async_copy(relp.at[idxr.at[cs]], rpb, sem).wait()

    def compute(hb, hpb, rpb, rb, ob):
        @plsc.parallel_loop(0, _CH, unroll=4)
        def row(i):
            hv = [hb[i, pl.ds(d * 16, 16)] for d in range(_LANES)]
            hpv = [hpb[i, pl.ds(d * 16, 16)] for d in range(_LANES)]
            acc = hv[0] * hpv[0]
            for d in range(1, _LANES):
                acc = acc + hv[d] * hpv[d]
            inner = jnp.sum(acc)
            tv = [rpb[i, pl.ds(d * 16, 16)] * inner + hv[d]
                  for d in range(_LANES)]
            nacc = tv[0] * tv[0]
            for d in range(1, _LANES):
                nacc = nacc + tv[d] * tv[d]
            inv1 = _rsqrt(jnp.maximum(jnp.sum(nacc), jnp.float32(1e-24)))
            uv = [tv[d] * inv1 + rb[i, pl.ds(d * 16, 16)]
                  for d in range(_LANES)]
            n2 = uv[0] * uv[0]
            for d in range(1, _LANES):
                n2 = n2 + uv[d] * uv[d]
            inv2 = _rsqrt(jnp.maximum(jnp.sum(n2), jnp.float32(1e-24)))
            for d in range(_LANES):
                ob[i, pl.ds(d * 16, 16)] = uv[d] * inv2

    fire(0, hbufA, hpbufA, rbufA, rpbufA, semA)

    def pair(p, _):
        c0 = 2 * p
        fire(c0 + 1, hbufB, hpbufB, rbufB, rpbufB, semB)
        drain(c0, hbufA, hpbufA, rbufA, rpbufA, semA)
        compute(hbufA, hpbufA, rpbufA, rbufA, obufA)
        wbA = pltpu.async_copy(
            obufA, out.at[pl.ds(base + c0 * _CH, _CH)], wsemA)

        @pl.when(p < _NPAIR - 1)
        def _():
            fire(c0 + 2, hbufA, hpbufA, rbufA, rpbufA, semA)

        drain(c0 + 1, hbufB, hpbufB, rbufB, rpbufB, semB)
        compute(hbufB, hpbufB, rpbufB, rbufB, obufB)
        wbB = pltpu.async_copy(
            obufB, out.at[pl.ds(base + (c0 + 1) * _CH, _CH)], wsemB)
        wbA.wait()
        wbB.wait()
        return 0

    lax.fori_loop(0, _NPAIR, pair, 0)


@functools.partial(jax.jit, donate_argnums=())
def _trans_d(heads_i32, rels_i32, inv_i32, entity_emb, entity_emb_p,
             rel2, rel_emb_p):
    mesh = plsc.VectorSubcoreMesh(
        core_axis_name="c", subcore_axis_name="s",
        num_cores=_NC, num_subcores=_NS)
    buf = lambda: pltpu.VMEM((_CH, _D), jnp.float32)
    return pl.kernel(
        _trans_d_body,
        out_type=jax.ShapeDtypeStruct((_B, _D), jnp.float32),
        mesh=mesh,
        compiler_params=pltpu.CompilerParams(needs_layout_passes=False),
        scratch_types=[
            pltpu.VMEM((_BPW,), jnp.int32),
            pltpu.VMEM((_BPW,), jnp.int32),
            pltpu.VMEM((_BPW,), jnp.int32),
            buf(), buf(), buf(), buf(), buf(),
            buf(), buf(), buf(), buf(), buf(),
            pltpu.SemaphoreType.DMA,
            pltpu.SemaphoreType.DMA,
            pltpu.SemaphoreType.DMA,
            pltpu.SemaphoreType.DMA,
        ],
    )(heads_i32, rels_i32, inv_i32, entity_emb, entity_emb_p,
      rel2, rel_emb_p)


def kernel(heads, relations, inverse, entity_emb, entity_emb_p,
           rel_emb, rel_emb_p):
    heads_i32 = heads.astype(jnp.int32)
    rels_i32 = relations.astype(jnp.int32)
    inv_i32 = inverse.astype(jnp.int32)
    rel2 = jnp.concatenate([rel_emb, -rel_emb], axis=0)
    return _trans_d(heads_i32, rels_i32, inv_i32, entity_emb,
                    entity_emb_p, rel2, rel_emb_p)
